# baseline (speedup 1.0000x reference)
import jax
import jax.numpy as jnp
from jax import lax
from jax.experimental import pallas as pl
from jax.experimental.pallas import tpu as pltpu
from jax.experimental.pallas import tpu_sc as plsc

_NC = 2
_NS = 16
_NW = _NC * _NS
_L = 16
_D = 32
_G = 4


def _mf_body(uidx_hbm, iidx_hbm, utabT_hbm, itabT_hbm, out_hbm,
             uidx_v, iidx_v, slab_u, slab_i, q_v, out_v,
             sem_u0, sem_u1, sem_u2, sem_i0, sem_i1, sem_i2):
    b_per_w = uidx_v.shape[0]
    wid = lax.axis_index("s") * _NC + lax.axis_index("c")
    base = wid * b_per_w

    pltpu.sync_copy(uidx_hbm.at[pl.ds(base, b_per_w)], uidx_v)
    pltpu.sync_copy(iidx_hbm.at[pl.ds(base, b_per_w)], iidx_v)

    lanes = lax.iota(jnp.int32, _L)
    sems_u = (sem_u0, sem_u1, sem_u2)
    sems_i = (sem_i0, sem_i1, sem_i2)

    def super_body(c, carry):
        b0 = c * _L
        u16 = uidx_v[pl.ds(b0, _L)]
        i16 = iidx_v[pl.ds(b0, _L)]
        utile = u16 >> 7
        itile = i16 >> 7
        ucol = u16 & 127
        icol = i16 & 127

        def fire(sub):
            p = sub % 3
            copies = []
            for k in range(_G):
                ka = sub * _G + k
                su = jnp.sum(jnp.where(lanes == ka, utile, 0))
                si = jnp.sum(jnp.where(lanes == ka, itile, 0))
                cu0 = pl.multiple_of(su * 128, 128)
                ci0 = pl.multiple_of(si * 128, 128)
                row0 = (p * _G + k) * _D
                for t1 in range(_D // 8):
                    copies.append(pltpu.async_copy(
                        utabT_hbm.at[pl.ds(t1 * 8, 8), pl.ds(cu0, 128)],
                        slab_u.at[pl.ds(row0 + t1 * 8, 8), :], sems_u[p]))
                    copies.append(pltpu.async_copy(
                        itabT_hbm.at[pl.ds(t1 * 8, 8), pl.ds(ci0, 128)],
                        slab_i.at[pl.ds(row0 + t1 * 8, 8), :], sems_i[p]))
            return copies

        def extract(sub):
            p = sub % 3
            for k in range(_G):
                ka = sub * _G + k
                kvec = jnp.full((_L,), ka, jnp.int32)
                cu = jnp.take_along_axis(ucol, kvec, axis=0)
                ci = jnp.take_along_axis(icol, kvec, axis=0)
                row0 = (p * _G + k) * _D
                u0 = plsc.load_gather(slab_u, [row0 + lanes, cu])
                u1 = plsc.load_gather(slab_u, [row0 + _L + lanes, cu])
                i0 = plsc.load_gather(slab_i, [row0 + lanes, ci])
                i1 = plsc.load_gather(slab_i, [row0 + _L + lanes, ci])
                q_v[pl.ds(ka * (_L + 1), _L)] = u0 * i0 + u1 * i1

        inflight = [fire(0), fire(1), fire(2)]
        for sub in range(_L // _G):
            for cp in inflight.pop(0):
                cp.wait()
            extract(sub)
            if sub + 3 < _L // _G:
                inflight.append(fire(sub + 3))
            else:
                inflight.append([])
        acc = jnp.zeros((_L,), jnp.float32)
        stride_lanes = lanes * (_L + 1)
        for j in range(_L):
            acc = acc + plsc.load_gather(q_v, [stride_lanes + j])
        out_v[pl.ds(b0, _L)] = 1.0 / (1.0 + jnp.exp(-acc))
        return carry

    lax.fori_loop(0, b_per_w // _L, super_body, 0)
    pltpu.sync_copy(out_v, out_hbm.at[pl.ds(base, b_per_w)])


def kernel(user_indices, item_indices, user_table, item_table):
    B = user_indices.shape[0]
    assert B % (_NW * _L) == 0
    assert user_table.shape[1] == _D
    b_per_w = B // _NW
    mesh = plsc.VectorSubcoreMesh(core_axis_name="c", subcore_axis_name="s",
                                  num_cores=_NC, num_subcores=_NS)
    run = pl.kernel(
        _mf_body,
        out_type=jax.ShapeDtypeStruct((B,), jnp.float32),
        mesh=mesh,
        compiler_params=pltpu.CompilerParams(needs_layout_passes=False),
        scratch_types=[
            pltpu.VMEM((b_per_w,), jnp.int32),
            pltpu.VMEM((b_per_w,), jnp.int32),
            pltpu.VMEM((3 * _G * _D, 128), jnp.float32),
            pltpu.VMEM((3 * _G * _D, 128), jnp.float32),
            pltpu.VMEM((_L * (_L + 1),), jnp.float32),
            pltpu.VMEM((b_per_w,), jnp.float32),
            pltpu.SemaphoreType.DMA,
            pltpu.SemaphoreType.DMA,
            pltpu.SemaphoreType.DMA,
            pltpu.SemaphoreType.DMA,
            pltpu.SemaphoreType.DMA,
            pltpu.SemaphoreType.DMA,
        ],
    )
    return run(user_indices, item_indices, user_table.T, item_table.T)
